# Initial kernel scaffold; baseline (speedup 1.0000x reference)
import jax
import jax.numpy as jnp
from jax.experimental import pallas as pl


def kernel(x, adj, W_in, b_in, Wm1, bm1, Wm2, bm2, Wm3, bm3, Wr1m, br1m, Wr2m, br2m, Wr1v, br1v, Wr2v, br2v):
    raise NotImplementedError("write your pallas kernel here")


# speedup vs baseline: 1.0855x; 1.0855x over previous
import functools

import jax
import jax.numpy as jnp
from jax.experimental import pallas as pl
from jax.experimental.pallas import tpu as pltpu

BLK = 512


def _body(adj_ref, x_ref, win_ref, bin_ref, wm1a_ref, wm1b_ref, bm1_ref,
          wm2_ref, bm2_ref, wm3_ref, bm3_ref,
          wr1m_ref, br1m_ref, wr2m_ref, br2m_ref,
          wr1v_ref, br1v_ref, wr2v_ref, br2v_ref,
          zm_ref, zlv_ref,
          h_scr, dinv_scr, pool_scr, *, n_nodes):
    t = pl.program_id(0)
    b = pl.program_id(1)
    i = pl.program_id(2)
    nb = pl.num_programs(2)
    rows = pl.ds(i * BLK, BLK)

    a = adj_ref[0]

    @pl.when(jnp.logical_and(t == 0, i == 0))
    def _():
        xb = x_ref[b]
        h0 = jnp.tanh(
            jnp.dot(xb, win_ref[...], preferred_element_type=jnp.float32)
            + bin_ref[...])
        h_scr[0, b] = h0
        pool_scr[b] = jnp.zeros_like(pool_scr[b])

    @pl.when(t == 0)
    def _():
        s = jnp.sum(a, axis=1, keepdims=True)
        dinv_scr[b, rows] = 1.0 / jnp.maximum(s, 1.0)

    src = t % 2
    dst = 1 - src

    h_all = h_scr[src, b]
    m = jnp.dot(a, h_all, preferred_element_type=jnp.float32)
    m = m * dinv_scr[b, rows]
    h_blk = h_scr[src, b, rows]

    u = jnp.dot(h_blk, wm1a_ref[...], preferred_element_type=jnp.float32)
    u = u + jnp.dot(m, wm1b_ref[...], preferred_element_type=jnp.float32)
    u = jax.nn.relu(u + bm1_ref[...])
    u = jax.nn.relu(
        jnp.dot(u, wm2_ref[...], preferred_element_type=jnp.float32)
        + bm2_ref[...])
    u = jnp.dot(u, wm3_ref[...], preferred_element_type=jnp.float32) + bm3_ref[...]
    h_new = h_blk + u

    @pl.when(t < 2)
    def _():
        h_scr[dst, b, rows] = h_new

    @pl.when(t == 2)
    def _():
        pool_scr[b] = pool_scr[b] + jnp.sum(h_new, axis=0)

    @pl.when(jnp.logical_and(t == 2, i == nb - 1))
    def _():
        pool = (pool_scr[b] * (1.0 / n_nodes)).reshape(1, -1)

        hm = jax.nn.relu(
            jnp.dot(pool, wr1m_ref[...], preferred_element_type=jnp.float32)
            + br1m_ref[...])
        zm = jnp.dot(hm, wr2m_ref[...], preferred_element_type=jnp.float32) + br2m_ref[...]
        zm_ref[b] = zm.reshape(-1)

        hv = jax.nn.relu(
            jnp.dot(pool, wr1v_ref[...], preferred_element_type=jnp.float32)
            + br1v_ref[...])
        zlv = jnp.dot(hv, wr2v_ref[...], preferred_element_type=jnp.float32) + br2v_ref[...]
        zlv_ref[b] = zlv.reshape(-1)


def kernel(x, adj, W_in, b_in, Wm1, bm1, Wm2, bm2, Wm3, bm3,
           Wr1m, br1m, Wr2m, br2m, Wr1v, br1v, Wr2v, br2v):
    B, N, D_IN = x.shape
    D_H = W_in.shape[1]
    D_Z = Wr2m.shape[1]
    nb = N // BLK

    Wm1a, Wm1b = Wm1[:D_H], Wm1[D_H:]

    def full(arr):
        return pl.BlockSpec(arr.shape, lambda t, b, i: (0,) * arr.ndim)

    biases = [b_in, bm1, bm2, bm3, br1m, br2m, br1v, br2v]
    b_in, bm1, bm2, bm3, br1m, br2m, br1v, br2v = [
        v.reshape(1, -1) for v in biases]

    in_specs = [
        pl.BlockSpec((1, BLK, N), lambda t, b, i: (b, i, 0)),
        full(x),
        full(W_in), full(b_in),
        full(Wm1a), full(Wm1b), full(bm1),
        full(Wm2), full(bm2), full(Wm3), full(bm3),
        full(Wr1m), full(br1m), full(Wr2m), full(br2m),
        full(Wr1v), full(br1v), full(Wr2v), full(br2v),
    ]

    out = pl.pallas_call(
        functools.partial(_body, n_nodes=N),
        grid=(3, B, nb),
        in_specs=in_specs,
        out_specs=[
            pl.BlockSpec((B, D_Z), lambda t, b, i: (0, 0)),
            pl.BlockSpec((B, D_Z), lambda t, b, i: (0, 0)),
        ],
        out_shape=[
            jax.ShapeDtypeStruct((B, D_Z), jnp.float32),
            jax.ShapeDtypeStruct((B, D_Z), jnp.float32),
        ],
        scratch_shapes=[
            pltpu.VMEM((2, B, N, D_H), jnp.float32),
            pltpu.VMEM((B, N, 1), jnp.float32),
            pltpu.VMEM((B, D_H), jnp.float32),
        ],
        compiler_params=pltpu.CompilerParams(
            dimension_semantics=("arbitrary", "arbitrary", "arbitrary")),
    )(adj, x, W_in, b_in, Wm1a, Wm1b, bm1, Wm2, bm2, Wm3, bm3,
      Wr1m, br1m, Wr2m, br2m, Wr1v, br1v, Wr2v, br2v)
    return (out[0], out[1])


# speedup vs baseline: 1.0860x; 1.0005x over previous
import functools

import jax
import jax.numpy as jnp
from jax.experimental import pallas as pl
from jax.experimental.pallas import tpu as pltpu

BLK = 512


def _body(adj_ref, x_ref, win_ref, bin_ref, wm1a_ref, wm1b_ref, bm1_ref,
          wm2_ref, bm2_ref, wm3_ref, bm3_ref,
          wr1m_ref, br1m_ref, wr2m_ref, br2m_ref,
          wr1v_ref, br1v_ref, wr2v_ref, br2v_ref,
          zm_ref, zlv_ref,
          h_scr, dinv_scr, pool_scr, *, n_nodes):
    t = pl.program_id(0)
    b = pl.program_id(1)
    i = pl.program_id(2)
    nb = pl.num_programs(2)
    rows = pl.ds(i * BLK, BLK)

    a = adj_ref[0]

    @pl.when(jnp.logical_and(t == 0, i == 0))
    def _():
        xb = x_ref[b]
        h0 = jnp.tanh(
            jnp.dot(xb, win_ref[...], preferred_element_type=jnp.float32)
            + bin_ref[...])
        h_scr[0, b] = h0
        pool_scr[b] = jnp.zeros_like(pool_scr[b])

    @pl.when(t == 0)
    def _():
        s = jnp.sum(a, axis=1, keepdims=True)
        dinv_scr[b, rows] = 1.0 / jnp.maximum(s, 1.0)

    src = t % 2
    dst = 1 - src

    h_all = h_scr[src, b]
    m = jnp.dot(a.astype(jnp.bfloat16), h_all.astype(jnp.bfloat16),
                preferred_element_type=jnp.float32)
    m = m * dinv_scr[b, rows]
    h_blk = h_scr[src, b, rows]

    u = jnp.dot(h_blk, wm1a_ref[...], preferred_element_type=jnp.float32)
    u = u + jnp.dot(m, wm1b_ref[...], preferred_element_type=jnp.float32)
    u = jax.nn.relu(u + bm1_ref[...])
    u = jax.nn.relu(
        jnp.dot(u, wm2_ref[...], preferred_element_type=jnp.float32)
        + bm2_ref[...])
    u = jnp.dot(u, wm3_ref[...], preferred_element_type=jnp.float32) + bm3_ref[...]
    h_new = h_blk + u

    @pl.when(t < 2)
    def _():
        h_scr[dst, b, rows] = h_new

    @pl.when(t == 2)
    def _():
        pool_scr[b] = pool_scr[b] + jnp.sum(h_new, axis=0)

    @pl.when(jnp.logical_and(t == 2, i == nb - 1))
    def _():
        pool = (pool_scr[b] * (1.0 / n_nodes)).reshape(1, -1)

        hm = jax.nn.relu(
            jnp.dot(pool, wr1m_ref[...], preferred_element_type=jnp.float32)
            + br1m_ref[...])
        zm = jnp.dot(hm, wr2m_ref[...], preferred_element_type=jnp.float32) + br2m_ref[...]
        zm_ref[b] = zm.reshape(-1)

        hv = jax.nn.relu(
            jnp.dot(pool, wr1v_ref[...], preferred_element_type=jnp.float32)
            + br1v_ref[...])
        zlv = jnp.dot(hv, wr2v_ref[...], preferred_element_type=jnp.float32) + br2v_ref[...]
        zlv_ref[b] = zlv.reshape(-1)


def kernel(x, adj, W_in, b_in, Wm1, bm1, Wm2, bm2, Wm3, bm3,
           Wr1m, br1m, Wr2m, br2m, Wr1v, br1v, Wr2v, br2v):
    B, N, D_IN = x.shape
    D_H = W_in.shape[1]
    D_Z = Wr2m.shape[1]
    nb = N // BLK

    Wm1a, Wm1b = Wm1[:D_H], Wm1[D_H:]

    def full(arr):
        return pl.BlockSpec(arr.shape, lambda t, b, i: (0,) * arr.ndim)

    biases = [b_in, bm1, bm2, bm3, br1m, br2m, br1v, br2v]
    b_in, bm1, bm2, bm3, br1m, br2m, br1v, br2v = [
        v.reshape(1, -1) for v in biases]

    in_specs = [
        pl.BlockSpec((1, BLK, N), lambda t, b, i: (b, i, 0)),
        full(x),
        full(W_in), full(b_in),
        full(Wm1a), full(Wm1b), full(bm1),
        full(Wm2), full(bm2), full(Wm3), full(bm3),
        full(Wr1m), full(br1m), full(Wr2m), full(br2m),
        full(Wr1v), full(br1v), full(Wr2v), full(br2v),
    ]

    out = pl.pallas_call(
        functools.partial(_body, n_nodes=N),
        grid=(3, B, nb),
        in_specs=in_specs,
        out_specs=[
            pl.BlockSpec((B, D_Z), lambda t, b, i: (0, 0)),
            pl.BlockSpec((B, D_Z), lambda t, b, i: (0, 0)),
        ],
        out_shape=[
            jax.ShapeDtypeStruct((B, D_Z), jnp.float32),
            jax.ShapeDtypeStruct((B, D_Z), jnp.float32),
        ],
        scratch_shapes=[
            pltpu.VMEM((2, B, N, D_H), jnp.float32),
            pltpu.VMEM((B, N, 1), jnp.float32),
            pltpu.VMEM((B, D_H), jnp.float32),
        ],
        compiler_params=pltpu.CompilerParams(
            dimension_semantics=("arbitrary", "arbitrary", "arbitrary")),
    )(adj, x, W_in, b_in, Wm1a, Wm1b, bm1, Wm2, bm2, Wm3, bm3,
      Wr1m, br1m, Wr2m, br2m, Wr1v, br1v, Wr2v, br2v)
    return (out[0], out[1])
